# Initial kernel scaffold; baseline (speedup 1.0000x reference)
import jax
import jax.numpy as jnp
from jax.experimental import pallas as pl


def kernel(from_up, from_down, gemm_edges, W_up, b_up, W1, b1, W2, b2):
    raise NotImplementedError("write your pallas kernel here")


# speedup vs baseline: 2.5932x; 2.5932x over previous
import jax
import jax.numpy as jnp
from jax import lax
from jax.experimental import pallas as pl
from jax.experimental.pallas import tpu as pltpu
from jax.experimental.pallas import tpu_sc as plsc

B = 4
E = 80000
M = B * E
C = 128

NW = 32
PER_W = M // NW
KCH = 80

BLK = 800
NEB = E // BLK


def _sc_gather_body(table, i1, i2, i3, i4, o1, o2, o3, o4,
                    iv1, iv2, iv3, iv4, bv1, bv2, bv3, bv4, sem):
    wid = lax.axis_index("s") * 2 + lax.axis_index("c")
    base0 = wid * PER_W
    ivs = (iv1, iv2, iv3, iv4)
    bvs = (bv1, bv2, bv3, bv4)
    ihs = (i1, i2, i3, i4)
    ohs = (o1, o2, o3, o4)

    def chunk(ci, carry):
        base = base0 + ci * KCH
        for iv, ih in zip(ivs, ihs):
            pltpu.sync_copy(ih.at[pl.ds(base, KCH)], iv)
        cps = [pltpu.async_copy(table.at[iv], bv, sem)
               for iv, bv in zip(ivs, bvs)]
        for cp in cps:
            cp.wait()
        for bv, oh in zip(bvs, ohs):
            pltpu.sync_copy(bv, oh.at[pl.ds(base, KCH)])
        return carry

    lax.fori_loop(0, PER_W // KCH, chunk, 0)


def _make_sc_gather():
    mesh = plsc.VectorSubcoreMesh(core_axis_name="c", subcore_axis_name="s")
    out_t = [jax.ShapeDtypeStruct((M, C), jnp.float32)] * 4
    scratch = ([pltpu.VMEM((KCH,), jnp.int32)] * 4
               + [pltpu.VMEM((KCH, C), jnp.float32)] * 4
               + [pltpu.SemaphoreType.DMA])
    return pl.kernel(_sc_gather_body, mesh=mesh, out_type=out_t,
                     scratch_types=scratch)


def _combo(a1, a2, a3, a4):
    s13 = a1[...] + a3[...]
    s24 = a2[...] + a4[...]
    d13 = jnp.abs(a1[...] - a3[...])
    d24 = jnp.abs(a2[...] - a4[...])
    return [s13, s24, d13, d24]


def _conv1_body(f0, a1, a2, a3, a4, w, bias, y):
    g = jnp.concatenate([f0[...]] + _combo(a1, a2, a3, a4),
                        axis=1).astype(jnp.bfloat16)
    y[...] = jnp.dot(g, w[...], preferred_element_type=jnp.float32) + bias[...]


def _stats_epilogue(i, y, acc1, acc2, scale, shift):
    @pl.when(i == 0)
    def _():
        acc1[...] = jnp.zeros_like(acc1)
        acc2[...] = jnp.zeros_like(acc2)

    acc1[...] += jnp.sum(y, axis=0, keepdims=True)
    acc2[...] += jnp.sum(y * y, axis=0, keepdims=True)

    @pl.when(i == NEB - 1)
    def _():
        mean = acc1[...] * (1.0 / E)
        var = acc2[...] * (1.0 / E) - mean * mean
        rstd = lax.rsqrt(var + 1e-5)
        scale[...] = rstd.reshape(1, 1, C)
        shift[...] = (-mean * rstd).reshape(1, 1, C)


def _conv2_body(y1r, fdr, g1, g2, g3, g4, h1, h2, h3, h4, w, bias,
                y2, scale, shift, acc1, acc2):
    i = pl.program_id(1)
    g = jnp.concatenate([y1r[...], fdr[...]] + _combo(g1, g2, g3, g4)
                        + _combo(h1, h2, h3, h4), axis=1).astype(jnp.bfloat16)
    y = jnp.dot(g, w[...], preferred_element_type=jnp.float32) + bias[...]
    y2[...] = y
    _stats_epilogue(i, y, acc1, acc2, scale, shift)


def _norm_body(y2r, scale, shift, x1):
    x1[...] = jnp.maximum(y2r[...] * scale[...].reshape(1, C)
                          + shift[...].reshape(1, C), 0.0)


def _conv3_body(x1r, a1, a2, a3, a4, w, bias, y3, scale, shift, acc1, acc2):
    i = pl.program_id(1)
    g = jnp.concatenate([x1r[...]] + _combo(a1, a2, a3, a4),
                        axis=1).astype(jnp.bfloat16)
    y = jnp.dot(g, w[...], preferred_element_type=jnp.float32) + bias[...]
    y3[...] = y
    _stats_epilogue(i, y, acc1, acc2, scale, shift)


def _row_spec(nin):
    return pl.BlockSpec((BLK, C), lambda b, i: (b * NEB + i, 0))


def _full_spec(k):
    return pl.BlockSpec((k, C), lambda b, i: (0, 0))


def _stat_spec():
    return pl.BlockSpec((1, 1, C), lambda b, i: (b, 0, 0))


_STAT_SHAPE = jax.ShapeDtypeStruct((B, 1, C), jnp.float32)


def _conv1_call(fu, a, wc, bias):
    spec = pl.BlockSpec((BLK, C), lambda i: (i, 0))
    return pl.pallas_call(
        _conv1_body,
        grid=(M // BLK,),
        in_specs=[spec] * 5 + [pl.BlockSpec((5 * C, C), lambda i: (0, 0)),
                               pl.BlockSpec((1, C), lambda i: (0, 0))],
        out_specs=spec,
        out_shape=jax.ShapeDtypeStruct((M, C), jnp.float32),
    )(fu, *a, wc, bias)


def _conv2_call(y1, fd, ga, gb, wc, bias):
    return pl.pallas_call(
        _conv2_body,
        grid=(B, NEB),
        in_specs=[_row_spec(0)] * 10 + [_full_spec(10 * C),
                                        pl.BlockSpec((1, C), lambda b, i: (0, 0))],
        out_specs=[_row_spec(0), _stat_spec(), _stat_spec()],
        out_shape=[jax.ShapeDtypeStruct((M, C), jnp.float32),
                   _STAT_SHAPE, _STAT_SHAPE],
        scratch_shapes=[pltpu.VMEM((1, C), jnp.float32),
                        pltpu.VMEM((1, C), jnp.float32)],
    )(y1, fd, *ga, *gb, wc, bias)


def _norm_call(y2, scale, shift):
    return pl.pallas_call(
        _norm_body,
        grid=(B, NEB),
        in_specs=[_row_spec(0), _stat_spec(), _stat_spec()],
        out_specs=_row_spec(0),
        out_shape=jax.ShapeDtypeStruct((M, C), jnp.float32),
    )(y2, scale, shift)


def _conv3_call(x1, a, wc, bias):
    return pl.pallas_call(
        _conv3_body,
        grid=(B, NEB),
        in_specs=[_row_spec(0)] * 5 + [_full_spec(5 * C),
                                       pl.BlockSpec((1, C), lambda b, i: (0, 0))],
        out_specs=[_row_spec(0), _stat_spec(), _stat_spec()],
        out_shape=[jax.ShapeDtypeStruct((M, C), jnp.float32),
                   _STAT_SHAPE, _STAT_SHAPE],
        scratch_shapes=[pltpu.VMEM((1, C), jnp.float32),
                        pltpu.VMEM((1, C), jnp.float32)],
    )(x1, *a, wc, bias)


def _final_body(y3r, x1r, scale, shift, out):
    out[...] = jnp.maximum(y3r[...] * scale[...].reshape(1, C)
                           + shift[...].reshape(1, C) + x1r[...], 0.0)


def _final_call(y3, x1, scale, shift):
    return pl.pallas_call(
        _final_body,
        grid=(B, NEB),
        in_specs=[_row_spec(0), _row_spec(0), _stat_spec(), _stat_spec()],
        out_specs=_row_spec(0),
        out_shape=jax.ShapeDtypeStruct((M, C), jnp.float32),
    )(y3, x1, scale, shift)


def kernel(from_up, from_down, gemm_edges, W_up, b_up, W1, b1, W2, b2):
    fu = from_up.transpose(0, 2, 1).reshape(M, C)
    fd = from_down.transpose(0, 2, 1).reshape(M, C)
    ge = gemm_edges.astype(jnp.int32) + (jnp.arange(B, dtype=jnp.int32) * E)[:, None, None]
    idx = [ge[:, :, j].reshape(M) for j in range(4)]

    def wcat(W, cols):
        return jnp.concatenate([W[:, cs, k].T for (cs, k) in cols],
                               axis=0).astype(jnp.bfloat16)

    full = slice(0, C)
    lo, hi = slice(0, C), slice(C, 2 * C)
    wc1 = wcat(W_up, [(full, 0), (full, 1), (full, 2), (full, 3), (full, 4)])
    wc2 = wcat(W1, [(lo, 0), (hi, 0), (lo, 1), (lo, 2), (lo, 3), (lo, 4),
                    (hi, 1), (hi, 2), (hi, 3), (hi, 4)])
    wc3 = wcat(W2, [(full, 0), (full, 1), (full, 2), (full, 3), (full, 4)])

    gather = _make_sc_gather()

    a = gather(fu, *idx)
    y1 = _conv1_call(fu, a, wc1, b_up.reshape(1, C))
    ga = gather(y1, *idx)
    gb = gather(fd, *idx)
    y2, scale2, shift2 = _conv2_call(y1, fd, ga, gb, wc2, b1.reshape(1, C))
    x1 = _norm_call(y2, scale2, shift2)
    da = gather(x1, *idx)
    y3, scale3, shift3 = _conv3_call(x1, da, wc3, b2.reshape(1, C))
    out = _final_call(y3, x1, scale3, shift3)
    return out.reshape(B, E, C).transpose(0, 2, 1)
